# baseline (speedup 1.0000x reference)
import functools

import jax
import jax.numpy as jnp
from jax import lax
from jax.experimental import pallas as pl
from jax.experimental.pallas import tpu as pltpu
from jax.experimental.pallas import tpu_sc as plsc

_NUM_TAGS = 1000
_TAG_DIM = 32
_BATCH = 16384
_SEQ_LEN = 200
_NW = 32
_BW = _BATCH // _NW
_SBLK = 8
_N_SBLK = _SEQ_LEN // _SBLK
_LANES = 16
_NGRP = _BW // _LANES
_NOUT = 4


def _make_kernel():
    mesh = plsc.VectorSubcoreMesh(core_axis_name="c", subcore_axis_name="s")

    @functools.partial(
        pl.kernel,
        mesh=mesh,
        compiler_params=pltpu.CompilerParams(
            needs_layout_passes=False, use_tc_tiling_on_sc=True
        ),
        out_type=jax.ShapeDtypeStruct((_SEQ_LEN, _TAG_DIM, _BATCH),
                                      jnp.float32),
        scratch_types=[
            pltpu.VMEM((_TAG_DIM * _NUM_TAGS,), jnp.float32),
            pltpu.VMEM((_SBLK, _BW), jnp.int32),
            pltpu.VMEM((_SBLK, _BW), jnp.int32),
            pltpu.VMEM((1, _TAG_DIM, _BW), jnp.float32),
            pltpu.VMEM((1, _TAG_DIM, _BW), jnp.float32),
            pltpu.VMEM((1, _TAG_DIM, _BW), jnp.float32),
            pltpu.VMEM((1, _TAG_DIM, _BW), jnp.float32),
            pltpu.SemaphoreType.DMA,
            pltpu.SemaphoreType.DMA,
            pltpu.SemaphoreType.DMA,
            pltpu.SemaphoreType.DMA,
            pltpu.SemaphoreType.DMA,
            pltpu.SemaphoreType.DMA,
        ],
    )
    def k(idxt_hbm, tabt_hbm, out_hbm, tab_v, idx_v0, idx_v1, row_v0,
          row_v1, row_v2, row_v3, si0, si1, so0, so1, so2, so3):
        wid = lax.axis_index("s") * 2 + lax.axis_index("c")
        b0 = wid * _BW
        pltpu.sync_copy(tabt_hbm, tab_v)

        idx_bufs = (idx_v0, idx_v1)
        row_bufs = (row_v0, row_v1, row_v2, row_v3)
        sin = (si0, si1)
        sout = (so0, so1, so2, so3)

        def start_in(blk, q):
            pltpu.async_copy(
                idxt_hbm.at[pl.ds(blk * _SBLK, _SBLK), pl.ds(b0, _BW)],
                idx_bufs[q], sin[q],
            )

        def wait_in(q):
            pltpu.make_async_copy(
                idxt_hbm.at[pl.ds(0, _SBLK), pl.ds(0, _BW)],
                idx_bufs[q], sin[q],
            ).wait()

        def start_out(s, p):
            pltpu.async_copy(
                row_bufs[p],
                out_hbm.at[pl.ds(s, 1), :, pl.ds(b0, _BW)],
                sout[p],
            )

        def wait_out(p):
            pltpu.make_async_copy(
                row_bufs[p],
                out_hbm.at[pl.ds(0, 1), :, pl.ds(0, _BW)],
                sout[p],
            ).wait()

        def compute(q, sl, p):
            idx_ref = idx_bufs[q]
            row_ref = row_bufs[p]

            @plsc.parallel_loop(0, _NGRP, unroll=2)
            def grp(bg):
                idxv = idx_ref[sl, pl.ds(bg * _LANES, _LANES)]
                vals = [
                    plsc.load_gather(tab_v, [idxv + d * _NUM_TAGS])
                    for d in range(_TAG_DIM)
                ]
                for d in range(_TAG_DIM):
                    row_ref[0, d, pl.ds(bg * _LANES, _LANES)] = vals[d]

        def sblock(blk, q):
            wait_in(q)

            def squad(s4, carry):
                for p in range(_NOUT):
                    sl = s4 * _NOUT + p
                    wait_out(p)
                    compute(q, sl, p)
                    start_out(blk * _SBLK + sl, p)
                return carry

            lax.fori_loop(0, _SBLK // _NOUT, squad, 0)

        start_in(0, 0)
        start_in(1, 1)
        for p in range(_NOUT):
            start_out(p, p)

        sblock(0, 0)
        start_in(2, 0)

        def step(t, carry):
            for q in (1, 0):
                blk = 2 * t + 1 + (1 - q)
                sblock(blk, q)
                start_in(blk + 2, q)
            return carry

        lax.fori_loop(0, _N_SBLK // 2 - 1, step, 0)

        sblock(_N_SBLK - 2, 1)
        sblock(_N_SBLK - 1, 0)
        for p in range(_NOUT):
            wait_out(p)

    return k


_gather_kernel = _make_kernel()


def kernel(tag_indices, tag_embeddings):
    idxt = tag_indices.T.astype(jnp.int32)
    tabt = tag_embeddings.T.reshape(-1)
    out = _gather_kernel(idxt, tabt)
    return jnp.transpose(out, (2, 0, 1))


# speedup vs baseline: 1.4664x; 1.4664x over previous
import functools

import jax
import jax.numpy as jnp
from jax import lax
from jax.experimental import pallas as pl
from jax.experimental.pallas import tpu as pltpu
from jax.experimental.pallas import tpu_sc as plsc

_NUM_TAGS = 1000
_TAG_DIM = 32
_BATCH = 16384
_SEQ_LEN = 200
_NW = 32
_BW = _BATCH // _NW
_SBLK = 8
_N_SBLK = _SEQ_LEN // _SBLK
_LANES = 16
_NGRP = _BW // _LANES
_NOUT = 4


def _make_kernel():
    mesh = plsc.VectorSubcoreMesh(core_axis_name="c", subcore_axis_name="s")

    @functools.partial(
        pl.kernel,
        mesh=mesh,
        compiler_params=pltpu.CompilerParams(
            needs_layout_passes=False, use_tc_tiling_on_sc=True
        ),
        out_type=jax.ShapeDtypeStruct((_SEQ_LEN, _TAG_DIM, _BATCH),
                                      jnp.float32),
        scratch_types=[
            pltpu.VMEM((_TAG_DIM * _NUM_TAGS,), jnp.float32),
            pltpu.VMEM((_SBLK, _BW), jnp.int32),
            pltpu.VMEM((_SBLK, _BW), jnp.int32),
            pltpu.VMEM((1, _TAG_DIM, _BW), jnp.float32),
            pltpu.VMEM((1, _TAG_DIM, _BW), jnp.float32),
            pltpu.VMEM((1, _TAG_DIM, _BW), jnp.float32),
            pltpu.VMEM((1, _TAG_DIM, _BW), jnp.float32),
            pltpu.SemaphoreType.DMA,
            pltpu.SemaphoreType.DMA,
            pltpu.SemaphoreType.DMA,
            pltpu.SemaphoreType.DMA,
            pltpu.SemaphoreType.DMA,
            pltpu.SemaphoreType.DMA,
        ],
    )
    def k(idxt_hbm, tabt_hbm, out_hbm, tab_v, idx_v0, idx_v1, row_v0,
          row_v1, row_v2, row_v3, si0, si1, so0, so1, so2, so3):
        wid = lax.axis_index("s") * 2 + lax.axis_index("c")
        b0 = wid * _BW
        pltpu.sync_copy(tabt_hbm, tab_v)

        idx_bufs = (idx_v0, idx_v1)
        row_bufs = (row_v0, row_v1, row_v2, row_v3)
        sin = (si0, si1)
        sout = (so0, so1, so2, so3)

        def start_in(blk, q):
            pltpu.async_copy(
                idxt_hbm.at[pl.ds(blk * _SBLK, _SBLK), pl.ds(b0, _BW)],
                idx_bufs[q], sin[q],
            )

        def wait_in(q):
            pltpu.make_async_copy(
                idxt_hbm.at[pl.ds(0, _SBLK), pl.ds(0, _BW)],
                idx_bufs[q], sin[q],
            ).wait()

        def start_out(s, p):
            pltpu.async_copy(
                row_bufs[p],
                out_hbm.at[pl.ds(s, 1), :, pl.ds(b0, _BW)],
                sout[p],
            )

        def wait_out(p):
            pltpu.make_async_copy(
                row_bufs[p],
                out_hbm.at[pl.ds(0, 1), :, pl.ds(0, _BW)],
                sout[p],
            ).wait()

        def compute(q, sl, p):
            idx_ref = idx_bufs[q]
            row_ref = row_bufs[p]

            @plsc.parallel_loop(0, _NGRP)
            def grp(bg):
                idxv = idx_ref[sl, pl.ds(bg * _LANES, _LANES)]
                for d0 in range(0, _TAG_DIM, 16):
                    vals = [
                        plsc.load_gather(tab_v, [idxv + d * _NUM_TAGS])
                        for d in range(d0, d0 + 16)
                    ]
                    for i, d in enumerate(range(d0, d0 + 16)):
                        row_ref[0, d, pl.ds(bg * _LANES, _LANES)] = vals[i]

        def sblock(blk, q):
            wait_in(q)

            def squad(s4, carry):
                for p in range(_NOUT):
                    sl = s4 * _NOUT + p
                    wait_out(p)
                    compute(q, sl, p)
                    start_out(blk * _SBLK + sl, p)
                return carry

            lax.fori_loop(0, _SBLK // _NOUT, squad, 0)

        start_in(0, 0)
        start_in(1, 1)
        for p in range(_NOUT):
            start_out(p, p)

        sblock(0, 0)
        start_in(2, 0)

        def step(t, carry):
            for q in (1, 0):
                blk = 2 * t + 1 + (1 - q)
                sblock(blk, q)
                start_in(blk + 2, q)
            return carry

        lax.fori_loop(0, _N_SBLK // 2 - 1, step, 0)

        sblock(_N_SBLK - 2, 1)
        sblock(_N_SBLK - 1, 0)
        for p in range(_NOUT):
            wait_out(p)

    return k


_gather_kernel = _make_kernel()


def kernel(tag_indices, tag_embeddings):
    idxt = tag_indices.T.astype(jnp.int32)
    tabt = tag_embeddings.T.reshape(-1)
    out = _gather_kernel(idxt, tabt)
    return jnp.transpose(out, (2, 0, 1))


# speedup vs baseline: 1.5801x; 1.0776x over previous
import functools

import jax
import jax.numpy as jnp
from jax import lax
from jax.experimental import pallas as pl
from jax.experimental.pallas import tpu as pltpu
from jax.experimental.pallas import tpu_sc as plsc

_NUM_TAGS = 1000
_TAG_DIM = 32
_BATCH = 16384
_SEQ_LEN = 200
_NW = 32
_BW = _BATCH // _NW
_SBLK = 8
_N_SBLK = _SEQ_LEN // _SBLK
_LANES = 16
_NGRP = _BW // _LANES
_NOUT = 4


def _make_kernel():
    mesh = plsc.VectorSubcoreMesh(core_axis_name="c", subcore_axis_name="s")

    @functools.partial(
        pl.kernel,
        mesh=mesh,
        compiler_params=pltpu.CompilerParams(
            needs_layout_passes=False, use_tc_tiling_on_sc=True
        ),
        out_type=jax.ShapeDtypeStruct((_SEQ_LEN, _TAG_DIM, _BATCH),
                                      jnp.float32),
        scratch_types=[
            pltpu.VMEM((_TAG_DIM * _NUM_TAGS,), jnp.float32),
            pltpu.VMEM((_SBLK, _BW), jnp.int32),
            pltpu.VMEM((_SBLK, _BW), jnp.int32),
            pltpu.VMEM((1, _TAG_DIM, _BW), jnp.float32),
            pltpu.VMEM((1, _TAG_DIM, _BW), jnp.float32),
            pltpu.VMEM((1, _TAG_DIM, _BW), jnp.float32),
            pltpu.VMEM((1, _TAG_DIM, _BW), jnp.float32),
            pltpu.SemaphoreType.DMA,
            pltpu.SemaphoreType.DMA,
            pltpu.SemaphoreType.DMA,
            pltpu.SemaphoreType.DMA,
            pltpu.SemaphoreType.DMA,
            pltpu.SemaphoreType.DMA,
        ],
    )
    def k(idxt_hbm, tabt_hbm, out_hbm, tab_v, idx_v0, idx_v1, row_v0,
          row_v1, row_v2, row_v3, si0, si1, so0, so1, so2, so3):
        wid = lax.axis_index("s") * 2 + lax.axis_index("c")
        b0 = wid * _BW
        pltpu.sync_copy(tabt_hbm, tab_v)

        idx_bufs = (idx_v0, idx_v1)
        row_bufs = (row_v0, row_v1, row_v2, row_v3)
        sin = (si0, si1)
        sout = (so0, so1, so2, so3)

        def start_in(blk, q):
            pltpu.async_copy(
                idxt_hbm.at[pl.ds(blk * _SBLK, _SBLK), pl.ds(b0, _BW)],
                idx_bufs[q], sin[q],
            )

        def wait_in(q):
            pltpu.make_async_copy(
                idxt_hbm.at[pl.ds(0, _SBLK), pl.ds(0, _BW)],
                idx_bufs[q], sin[q],
            ).wait()

        def start_out(s, p):
            pltpu.async_copy(
                row_bufs[p],
                out_hbm.at[pl.ds(s, 1), :, pl.ds(b0, _BW)],
                sout[p],
            )

        def wait_out(p):
            pltpu.make_async_copy(
                row_bufs[p],
                out_hbm.at[pl.ds(0, 1), :, pl.ds(0, _BW)],
                sout[p],
            ).wait()

        def compute(q, sl, p):
            idx_ref = idx_bufs[q]
            row_ref = row_bufs[p]

            @plsc.parallel_loop(0, _NGRP)
            def grp(bg):
                idxv = idx_ref[sl, pl.ds(bg * _LANES, _LANES)]
                vals = [
                    plsc.load_gather(tab_v, [idxv + d * _NUM_TAGS])
                    for d in range(_TAG_DIM)
                ]
                for d in range(_TAG_DIM):
                    row_ref[0, d, pl.ds(bg * _LANES, _LANES)] = vals[d]

        def sblock(blk, q):
            wait_in(q)

            def squad(s4, carry):
                for p in range(_NOUT):
                    sl = s4 * _NOUT + p
                    wait_out(p)
                    compute(q, sl, p)
                    start_out(blk * _SBLK + sl, p)
                return carry

            lax.fori_loop(0, _SBLK // _NOUT, squad, 0)

        start_in(0, 0)
        start_in(1, 1)
        for p in range(_NOUT):
            start_out(p, p)

        sblock(0, 0)
        start_in(2, 0)

        def step(t, carry):
            for q in (1, 0):
                blk = 2 * t + 1 + (1 - q)
                sblock(blk, q)
                start_in(blk + 2, q)
            return carry

        lax.fori_loop(0, _N_SBLK // 2 - 1, step, 0)

        sblock(_N_SBLK - 2, 1)
        sblock(_N_SBLK - 1, 0)
        for p in range(_NOUT):
            wait_out(p)

    return k


_gather_kernel = _make_kernel()


def kernel(tag_indices, tag_embeddings):
    idxt = tag_indices.T.astype(jnp.int32)
    tabt = tag_embeddings.T.reshape(-1)
    out = _gather_kernel(idxt, tabt)
    return jnp.transpose(out, (2, 0, 1))
